# baseline (speedup 1.0000x reference)
import jax
import jax.numpy as jnp
from jax.experimental import pallas as pl
from jax.experimental.pallas import tpu as pltpu

_QL = 4
_TK = 3


def _body(mq, pq, tae, ego, ptm, pem, ete,
          out1, out2, out6, out3, out4, out5, out7, out8, *sems):
    copies = []
    si = 0
    for q in range(_QL):
        copies.append(pltpu.make_async_copy(mq, out1.at[:, q], sems[si])); si += 1
        copies.append(pltpu.make_async_copy(pq, out2.at[:, q], sems[si])); si += 1
    for t in range(_TK):
        copies.append(pltpu.make_async_copy(tae.at[:, :, t, :], out6.at[:, t], sems[si])); si += 1
    for c in copies:
        c.start()

    ego_v = ego[...]
    for q in range(_QL):
        out3[:, q] = ego_v
    out4[...] = jnp.zeros(out4.shape, jnp.int32)

    ptm_v = ptm[...]
    for t in range(_TK):
        out5[:, t] = ((ptm_v >> (8 * t)) & 1).astype(jnp.int8)

    pem_v = pem[...]
    b0 = (pem_v >> 0) & 1
    b1 = (pem_v >> 8) & 1
    b2 = (pem_v >> 16) & 1
    for t, bt in enumerate((b0, b1, b2)):
        out7[:, t] = bt.astype(jnp.int8)

    all_true = (b0 + b1 + b2) == 3
    ff = jnp.where(b0 == 0, 0, jnp.where(b1 == 0, 1, 2))
    pe0 = ete[:, 0, 0]
    pe1 = ete[:, 0, 1]
    pe2 = ete[:, 0, 2]
    last = ete[:, 0, 3]
    tmp = jnp.where(ff == 0, pe0, jnp.where(ff == 1, pe1, pe2))
    for t, pet in enumerate((pe0, pe1, pe2)):
        val = jnp.where(all_true, last, jnp.where(t < ff, tmp, pet))
        out8[:, t, 0] = val

    for c in copies:
        c.wait()


def kernel(motion_query, plan_query, ego_status_feature, mask,
           temp_anchor_embed_forstate, temp_mask_forstate,
           ego_temp_anchor_embed_forstate, ego_temp_mask_forstate):
    del mask
    B, N, D = motion_query.shape
    P = plan_query.shape[1]

    ptm = jax.lax.bitcast_convert_type(
        temp_mask_forstate.astype(jnp.uint8), jnp.int32)
    pem = jax.lax.bitcast_convert_type(
        ego_temp_mask_forstate.astype(jnp.uint8), jnp.int32)

    any_spec = pl.BlockSpec(memory_space=pl.ANY)
    vmem_spec = pl.BlockSpec(memory_space=pltpu.VMEM)
    n_dma = 2 * _QL + _TK

    out1, out2, out6, out3, out4, out5, out7, out8 = pl.pallas_call(
        _body,
        in_specs=[any_spec, any_spec, any_spec,
                  vmem_spec, vmem_spec, vmem_spec, vmem_spec],
        out_specs=[any_spec, any_spec, any_spec,
                   vmem_spec, vmem_spec, vmem_spec, vmem_spec, vmem_spec],
        out_shape=[
            jax.ShapeDtypeStruct((B, _QL, N, D), jnp.float32),
            jax.ShapeDtypeStruct((B, _QL, P, D), jnp.float32),
            jax.ShapeDtypeStruct((B, _TK, N, D), jnp.float32),
            jax.ShapeDtypeStruct((B, _QL, 1, D), jnp.float32),
            jax.ShapeDtypeStruct((B, _QL), jnp.int32),
            jax.ShapeDtypeStruct((B, _TK, N), jnp.int8),
            jax.ShapeDtypeStruct((B, _TK, 1), jnp.int8),
            jax.ShapeDtypeStruct((B, _TK, 1, D), jnp.float32),
        ],
        scratch_shapes=[pltpu.SemaphoreType.DMA] * n_dma,
    )(motion_query, plan_query, temp_anchor_embed_forstate,
      ego_status_feature, ptm, pem, ego_temp_anchor_embed_forstate)

    return (out1, out2, out3, out4,
            out5.astype(bool), out6, out7.astype(bool), out8)


# speedup vs baseline: 11.0742x; 11.0742x over previous
import jax
import jax.numpy as jnp
from jax.experimental import pallas as pl
from jax.experimental.pallas import tpu as pltpu

_QL = 4
_TK = 3


def _body(mq, pq, tae, ego, ptm, pem, ete,
          out1, out2, out6, out3, out4, out5, out7, out8):
    out1[...] = mq[...]
    out2[...] = pq[...]
    out6[...] = tae[...]

    ego_v = ego[...]
    for q in range(_QL):
        out3[:, q] = ego_v
    out4[...] = jnp.zeros(out4.shape, jnp.int32)

    ptm_v = ptm[...]
    for t in range(_TK):
        out5[:, t] = ((ptm_v >> (8 * t)) & 1).astype(jnp.int8)

    pem_v = pem[...]
    b0 = (pem_v >> 0) & 1
    b1 = (pem_v >> 8) & 1
    b2 = (pem_v >> 16) & 1
    for t, bt in enumerate((b0, b1, b2)):
        out7[:, t] = bt.astype(jnp.int8)

    all_true = (b0 + b1 + b2) == 3
    ff = jnp.where(b0 == 0, 0, jnp.where(b1 == 0, 1, 2))
    pe0 = ete[:, 0]
    pe1 = ete[:, 1]
    pe2 = ete[:, 2]
    last = ete[:, 3]
    tmp = jnp.where(ff == 0, pe0, jnp.where(ff == 1, pe1, pe2))
    for t, pet in enumerate((pe0, pe1, pe2)):
        val = jnp.where(all_true, last, jnp.where(t < ff, tmp, pet))
        out8[:, t, 0] = val


def kernel(motion_query, plan_query, ego_status_feature, mask,
           temp_anchor_embed_forstate, temp_mask_forstate,
           ego_temp_anchor_embed_forstate, ego_temp_mask_forstate):
    del mask
    B, N, D = motion_query.shape
    P = plan_query.shape[1]
    sq = pl.squeezed

    ptm = jax.lax.bitcast_convert_type(
        temp_mask_forstate.astype(jnp.uint8), jnp.int32)
    pem = jax.lax.bitcast_convert_type(
        ego_temp_mask_forstate.astype(jnp.uint8), jnp.int32)
    ete = ego_temp_anchor_embed_forstate.reshape(B, _QL, D)
    tae = temp_anchor_embed_forstate.reshape(B, N, _QL * D)

    out1, out2, out6, out3, out4, out5, out7, out8 = pl.pallas_call(
        _body,
        grid=(B, _QL),
        in_specs=[
            pl.BlockSpec((sq, N, D), lambda b, t: (b, 0, 0)),
            pl.BlockSpec((sq, P, D), lambda b, t: (b, 0, 0)),
            pl.BlockSpec((sq, N, D),
                         lambda b, t: (b, 0, jnp.minimum(t, _TK - 1))),
            pl.BlockSpec((B, 1, D), lambda b, t: (0, 0, 0)),
            pl.BlockSpec((B, N), lambda b, t: (0, 0)),
            pl.BlockSpec((B, 1), lambda b, t: (0, 0)),
            pl.BlockSpec((B, _QL, D), lambda b, t: (0, 0, 0)),
        ],
        out_specs=[
            pl.BlockSpec((sq, sq, N, D), lambda b, t: (b, t, 0, 0)),
            pl.BlockSpec((sq, sq, P, D), lambda b, t: (b, t, 0, 0)),
            pl.BlockSpec((sq, sq, N, D),
                         lambda b, t: (b, jnp.minimum(t, _TK - 1), 0, 0)),
            pl.BlockSpec((B, _QL, 1, D), lambda b, t: (0, 0, 0, 0)),
            pl.BlockSpec((B, _QL), lambda b, t: (0, 0)),
            pl.BlockSpec((B, _TK, N), lambda b, t: (0, 0, 0)),
            pl.BlockSpec((B, _TK, 1), lambda b, t: (0, 0, 0)),
            pl.BlockSpec((B, _TK, 1, D), lambda b, t: (0, 0, 0, 0)),
        ],
        out_shape=[
            jax.ShapeDtypeStruct((B, _QL, N, D), jnp.float32),
            jax.ShapeDtypeStruct((B, _QL, P, D), jnp.float32),
            jax.ShapeDtypeStruct((B, _TK, N, D), jnp.float32),
            jax.ShapeDtypeStruct((B, _QL, 1, D), jnp.float32),
            jax.ShapeDtypeStruct((B, _QL), jnp.int32),
            jax.ShapeDtypeStruct((B, _TK, N), jnp.int8),
            jax.ShapeDtypeStruct((B, _TK, 1), jnp.int8),
            jax.ShapeDtypeStruct((B, _TK, 1, D), jnp.float32),
        ],
    )(motion_query, plan_query, tae,
      ego_status_feature, ptm, pem, ete)

    return (out1, out2, out3, out4,
            out5.astype(bool), out6, out7.astype(bool), out8)


# speedup vs baseline: 11.2863x; 1.0191x over previous
import functools

import jax
import jax.numpy as jnp
from jax import lax
from jax.experimental import pallas as pl
from jax.experimental.pallas import tpu as pltpu
from jax.experimental.pallas import tpu_sc as plsc

_QL = 4
_TK = 3
_NC = 2
_NS = 16
_NCH = 2


def _tc_body(mq, pq, ego, ptm, pem, ete,
             out1, out2, out3, out4, out5, out7, out8):
    out1[...] = mq[...]
    out2[...] = pq[...]

    ego_v = ego[...]
    for q in range(_QL):
        out3[:, q] = ego_v
    out4[...] = jnp.zeros(out4.shape, jnp.int32)

    ptm_v = ptm[...]
    for t in range(_TK):
        out5[:, t] = ((ptm_v >> (8 * t)) & 1).astype(jnp.int8)

    pem_v = pem[...]
    b0 = (pem_v >> 0) & 1
    b1 = (pem_v >> 8) & 1
    b2 = (pem_v >> 16) & 1
    for t, bt in enumerate((b0, b1, b2)):
        out7[:, t] = bt.astype(jnp.int8)

    all_true = (b0 + b1 + b2) == 3
    ff = jnp.where(b0 == 0, 0, jnp.where(b1 == 0, 1, 2))
    pe0 = ete[:, 0]
    pe1 = ete[:, 1]
    pe2 = ete[:, 2]
    last = ete[:, 3]
    tmp = jnp.where(ff == 0, pe0, jnp.where(ff == 1, pe1, pe2))
    for t, pet in enumerate((pe0, pe1, pe2)):
        val = jnp.where(all_true, last, jnp.where(t < ff, tmp, pet))
        out8[:, t, 0] = val


def _sc_gather(B, N, D):
    ch = D // _NCH
    n_items = B * _TK * _NCH
    n_workers = _NC * _NS
    per_worker = n_items // n_workers
    mesh = plsc.VectorSubcoreMesh(core_axis_name="c", subcore_axis_name="s")

    @functools.partial(
        pl.kernel, mesh=mesh,
        out_type=jax.ShapeDtypeStruct((B, _TK, N, D), jnp.float32),
        scratch_types=[pltpu.VMEM((N, D // _NCH), jnp.float32)],
    )
    def sc_copy(tae_hbm, out_hbm, buf):
        wid = lax.axis_index("s") * _NC + lax.axis_index("c")
        for i in range(per_worker):
            idx = wid * per_worker + i
            b = idx // (_TK * _NCH)
            r = idx % (_TK * _NCH)
            t = r // _NCH
            d0 = (r % _NCH) * ch
            pltpu.sync_copy(tae_hbm.at[b, :, pl.ds(t * D + d0, ch)], buf)
            pltpu.sync_copy(buf, out_hbm.at[b, t, :, pl.ds(d0, ch)])

    return sc_copy


def kernel(motion_query, plan_query, ego_status_feature, mask,
           temp_anchor_embed_forstate, temp_mask_forstate,
           ego_temp_anchor_embed_forstate, ego_temp_mask_forstate):
    del mask
    B, N, D = motion_query.shape
    P = plan_query.shape[1]
    sq = pl.squeezed

    ptm = jax.lax.bitcast_convert_type(
        temp_mask_forstate.astype(jnp.uint8), jnp.int32)
    pem = jax.lax.bitcast_convert_type(
        ego_temp_mask_forstate.astype(jnp.uint8), jnp.int32)
    ete = ego_temp_anchor_embed_forstate.reshape(B, _QL, D)
    tae = temp_anchor_embed_forstate.reshape(B, N, _QL * D)

    out6 = _sc_gather(B, N, D)(tae)

    out1, out2, out3, out4, out5, out7, out8 = pl.pallas_call(
        _tc_body,
        grid=(B, _QL),
        in_specs=[
            pl.BlockSpec((sq, N, D), lambda b, t: (b, 0, 0)),
            pl.BlockSpec((sq, P, D), lambda b, t: (b, 0, 0)),
            pl.BlockSpec((B, 1, D), lambda b, t: (0, 0, 0)),
            pl.BlockSpec((B, N), lambda b, t: (0, 0)),
            pl.BlockSpec((B, 1), lambda b, t: (0, 0)),
            pl.BlockSpec((B, _QL, D), lambda b, t: (0, 0, 0)),
        ],
        out_specs=[
            pl.BlockSpec((sq, sq, N, D), lambda b, t: (b, t, 0, 0)),
            pl.BlockSpec((sq, sq, P, D), lambda b, t: (b, t, 0, 0)),
            pl.BlockSpec((B, _QL, 1, D), lambda b, t: (0, 0, 0, 0)),
            pl.BlockSpec((B, _QL), lambda b, t: (0, 0)),
            pl.BlockSpec((B, _TK, N), lambda b, t: (0, 0, 0)),
            pl.BlockSpec((B, _TK, 1), lambda b, t: (0, 0, 0)),
            pl.BlockSpec((B, _TK, 1, D), lambda b, t: (0, 0, 0, 0)),
        ],
        out_shape=[
            jax.ShapeDtypeStruct((B, _QL, N, D), jnp.float32),
            jax.ShapeDtypeStruct((B, _QL, P, D), jnp.float32),
            jax.ShapeDtypeStruct((B, _QL, 1, D), jnp.float32),
            jax.ShapeDtypeStruct((B, _QL), jnp.int32),
            jax.ShapeDtypeStruct((B, _TK, N), jnp.int8),
            jax.ShapeDtypeStruct((B, _TK, 1), jnp.int8),
            jax.ShapeDtypeStruct((B, _TK, 1, D), jnp.float32),
        ],
    )(motion_query, plan_query, ego_status_feature, ptm, pem, ete)

    return (out1, out2, out3, out4,
            out5.astype(bool), out6, out7.astype(bool), out8)


# speedup vs baseline: 11.3029x; 1.0015x over previous
import functools

import jax
import jax.numpy as jnp
from jax import lax
from jax.experimental import pallas as pl
from jax.experimental.pallas import tpu as pltpu
from jax.experimental.pallas import tpu_sc as plsc

_QL = 4
_TK = 3
_NC = 2
_NS = 16
_NCH = 2


def _tc_body(mq, pq, ego, ptm, pem, ete,
             out1, out2, out3, out4, out5, out7, out8):
    out1[...] = mq[...]
    out2[...] = pq[...]

    @pl.when(jnp.logical_and(pl.program_id(0) == 0, pl.program_id(1) == 0))
    def _smalls():
        _small_outputs(ego, ptm, pem, ete, out3, out4, out5, out7, out8)


def _small_outputs(ego, ptm, pem, ete, out3, out4, out5, out7, out8):
    ego_v = ego[...]
    for q in range(_QL):
        out3[:, q] = ego_v
    out4[...] = jnp.zeros(out4.shape, jnp.int32)

    ptm_v = ptm[...]
    for t in range(_TK):
        out5[:, t] = ((ptm_v >> (8 * t)) & 1).astype(jnp.int8)

    pem_v = pem[...]
    b0 = (pem_v >> 0) & 1
    b1 = (pem_v >> 8) & 1
    b2 = (pem_v >> 16) & 1
    for t, bt in enumerate((b0, b1, b2)):
        out7[:, t] = bt.astype(jnp.int8)

    all_true = (b0 + b1 + b2) == 3
    ff = jnp.where(b0 == 0, 0, jnp.where(b1 == 0, 1, 2))
    pe0 = ete[:, 0]
    pe1 = ete[:, 1]
    pe2 = ete[:, 2]
    last = ete[:, 3]
    tmp = jnp.where(ff == 0, pe0, jnp.where(ff == 1, pe1, pe2))
    for t, pet in enumerate((pe0, pe1, pe2)):
        val = jnp.where(all_true, last, jnp.where(t < ff, tmp, pet))
        out8[:, t, 0] = val


def _sc_gather(B, N, D):
    ch = D // _NCH
    n_items = B * _TK * _NCH
    n_workers = _NC * _NS
    per_worker = n_items // n_workers
    mesh = plsc.VectorSubcoreMesh(core_axis_name="c", subcore_axis_name="s")

    @functools.partial(
        pl.kernel, mesh=mesh,
        out_type=jax.ShapeDtypeStruct((B, _TK, N, D), jnp.float32),
        scratch_types=[pltpu.VMEM((N, D // _NCH), jnp.float32)],
    )
    def sc_copy(tae_hbm, out_hbm, buf):
        wid = lax.axis_index("s") * _NC + lax.axis_index("c")
        for i in range(per_worker):
            idx = wid * per_worker + i
            b = idx // (_TK * _NCH)
            r = idx % (_TK * _NCH)
            t = r // _NCH
            d0 = (r % _NCH) * ch
            pltpu.sync_copy(tae_hbm.at[b, :, pl.ds(t * D + d0, ch)], buf)
            pltpu.sync_copy(buf, out_hbm.at[b, t, :, pl.ds(d0, ch)])

    return sc_copy


def kernel(motion_query, plan_query, ego_status_feature, mask,
           temp_anchor_embed_forstate, temp_mask_forstate,
           ego_temp_anchor_embed_forstate, ego_temp_mask_forstate):
    del mask
    B, N, D = motion_query.shape
    P = plan_query.shape[1]
    sq = pl.squeezed

    ptm = jax.lax.bitcast_convert_type(
        temp_mask_forstate.astype(jnp.uint8), jnp.int32)
    pem = jax.lax.bitcast_convert_type(
        ego_temp_mask_forstate.astype(jnp.uint8), jnp.int32)
    ete = ego_temp_anchor_embed_forstate.reshape(B, _QL, D)
    tae = temp_anchor_embed_forstate.reshape(B, N, _QL * D)

    out6 = _sc_gather(B, N, D)(tae)

    out1, out2, out3, out4, out5, out7, out8 = pl.pallas_call(
        _tc_body,
        grid=(B, _QL),
        in_specs=[
            pl.BlockSpec((sq, N, D), lambda b, t: (b, 0, 0)),
            pl.BlockSpec((sq, P, D), lambda b, t: (b, 0, 0)),
            pl.BlockSpec((B, 1, D), lambda b, t: (0, 0, 0)),
            pl.BlockSpec((B, N), lambda b, t: (0, 0)),
            pl.BlockSpec((B, 1), lambda b, t: (0, 0)),
            pl.BlockSpec((B, _QL, D), lambda b, t: (0, 0, 0)),
        ],
        out_specs=[
            pl.BlockSpec((sq, sq, N, D), lambda b, t: (b, t, 0, 0)),
            pl.BlockSpec((sq, sq, P, D), lambda b, t: (b, t, 0, 0)),
            pl.BlockSpec((B, _QL, 1, D), lambda b, t: (0, 0, 0, 0)),
            pl.BlockSpec((B, _QL), lambda b, t: (0, 0)),
            pl.BlockSpec((B, _TK, N), lambda b, t: (0, 0, 0)),
            pl.BlockSpec((B, _TK, 1), lambda b, t: (0, 0, 0)),
            pl.BlockSpec((B, _TK, 1, D), lambda b, t: (0, 0, 0, 0)),
        ],
        out_shape=[
            jax.ShapeDtypeStruct((B, _QL, N, D), jnp.float32),
            jax.ShapeDtypeStruct((B, _QL, P, D), jnp.float32),
            jax.ShapeDtypeStruct((B, _QL, 1, D), jnp.float32),
            jax.ShapeDtypeStruct((B, _QL), jnp.int32),
            jax.ShapeDtypeStruct((B, _TK, N), jnp.int8),
            jax.ShapeDtypeStruct((B, _TK, 1), jnp.int8),
            jax.ShapeDtypeStruct((B, _TK, 1, D), jnp.float32),
        ],
    )(motion_query, plan_query, ego_status_feature, ptm, pem, ete)

    return (out1, out2, out3, out4,
            out5.astype(bool), out6, out7.astype(bool), out8)


# speedup vs baseline: 12.2709x; 1.0856x over previous
import functools

import jax
import jax.numpy as jnp
from jax import lax
from jax.experimental import pallas as pl
from jax.experimental.pallas import tpu as pltpu
from jax.experimental.pallas import tpu_sc as plsc

_QL = 4
_TK = 3
_NC = 2
_NS = 16
_NCH = 2


def _tc_body(mq, pq, ego, ptm, pem, ete,
             out1, out2, out3, out4, out5, out7, out8):
    mq_v = mq[...]
    pq_v = pq[...]
    for q in range(_QL):
        out1[q] = mq_v
        out2[q] = pq_v

    @pl.when(pl.program_id(0) == 0)
    def _smalls():
        _small_outputs(ego, ptm, pem, ete, out3, out4, out5, out7, out8)


def _small_outputs(ego, ptm, pem, ete, out3, out4, out5, out7, out8):
    ego_v = ego[...]
    for q in range(_QL):
        out3[:, q] = ego_v
    out4[...] = jnp.zeros(out4.shape, jnp.int32)

    ptm_v = ptm[...]
    for t in range(_TK):
        out5[:, t] = ((ptm_v >> (8 * t)) & 1).astype(jnp.int8)

    pem_v = pem[...]
    b0 = (pem_v >> 0) & 1
    b1 = (pem_v >> 8) & 1
    b2 = (pem_v >> 16) & 1
    for t, bt in enumerate((b0, b1, b2)):
        out7[:, t] = bt.astype(jnp.int8)

    all_true = (b0 + b1 + b2) == 3
    ff = jnp.where(b0 == 0, 0, jnp.where(b1 == 0, 1, 2))
    pe0 = ete[:, 0]
    pe1 = ete[:, 1]
    pe2 = ete[:, 2]
    last = ete[:, 3]
    tmp = jnp.where(ff == 0, pe0, jnp.where(ff == 1, pe1, pe2))
    for t, pet in enumerate((pe0, pe1, pe2)):
        val = jnp.where(all_true, last, jnp.where(t < ff, tmp, pet))
        out8[:, t, 0] = val


def _sc_gather(B, N, D):
    ch = D // _NCH
    n_items = B * _TK * _NCH
    n_workers = _NC * _NS
    per_worker = n_items // n_workers
    mesh = plsc.VectorSubcoreMesh(core_axis_name="c", subcore_axis_name="s")

    @functools.partial(
        pl.kernel, mesh=mesh,
        out_type=jax.ShapeDtypeStruct((B, _TK, N, D), jnp.float32),
        scratch_types=[pltpu.VMEM((N, D // _NCH), jnp.float32)],
    )
    def sc_copy(tae_hbm, out_hbm, buf):
        wid = lax.axis_index("s") * _NC + lax.axis_index("c")
        for i in range(per_worker):
            idx = wid * per_worker + i
            b = idx // (_TK * _NCH)
            r = idx % (_TK * _NCH)
            t = r // _NCH
            d0 = (r % _NCH) * ch
            pltpu.sync_copy(tae_hbm.at[b, :, pl.ds(t * D + d0, ch)], buf)
            pltpu.sync_copy(buf, out_hbm.at[b, t, :, pl.ds(d0, ch)])

    return sc_copy


def kernel(motion_query, plan_query, ego_status_feature, mask,
           temp_anchor_embed_forstate, temp_mask_forstate,
           ego_temp_anchor_embed_forstate, ego_temp_mask_forstate):
    del mask
    B, N, D = motion_query.shape
    P = plan_query.shape[1]
    sq = pl.squeezed

    ptm = jax.lax.bitcast_convert_type(
        temp_mask_forstate.astype(jnp.uint8), jnp.int32)
    pem = jax.lax.bitcast_convert_type(
        ego_temp_mask_forstate.astype(jnp.uint8), jnp.int32)
    ete = ego_temp_anchor_embed_forstate.reshape(B, _QL, D)
    tae = temp_anchor_embed_forstate.reshape(B, N, _QL * D)

    out6 = _sc_gather(B, N, D)(tae)

    out1, out2, out3, out4, out5, out7, out8 = pl.pallas_call(
        _tc_body,
        grid=(B,),
        in_specs=[
            pl.BlockSpec((sq, N, D), lambda b: (b, 0, 0)),
            pl.BlockSpec((sq, P, D), lambda b: (b, 0, 0)),
            pl.BlockSpec((B, 1, D), lambda b: (0, 0, 0)),
            pl.BlockSpec((B, N), lambda b: (0, 0)),
            pl.BlockSpec((B, 1), lambda b: (0, 0)),
            pl.BlockSpec((B, _QL, D), lambda b: (0, 0, 0)),
        ],
        out_specs=[
            pl.BlockSpec((sq, _QL, N, D), lambda b: (b, 0, 0, 0)),
            pl.BlockSpec((sq, _QL, P, D), lambda b: (b, 0, 0, 0)),
            pl.BlockSpec((B, _QL, 1, D), lambda b: (0, 0, 0, 0)),
            pl.BlockSpec((B, _QL), lambda b: (0, 0)),
            pl.BlockSpec((B, _TK, N), lambda b: (0, 0, 0)),
            pl.BlockSpec((B, _TK, 1), lambda b: (0, 0, 0)),
            pl.BlockSpec((B, _TK, 1, D), lambda b: (0, 0, 0, 0)),
        ],
        out_shape=[
            jax.ShapeDtypeStruct((B, _QL, N, D), jnp.float32),
            jax.ShapeDtypeStruct((B, _QL, P, D), jnp.float32),
            jax.ShapeDtypeStruct((B, _QL, 1, D), jnp.float32),
            jax.ShapeDtypeStruct((B, _QL), jnp.int32),
            jax.ShapeDtypeStruct((B, _TK, N), jnp.int8),
            jax.ShapeDtypeStruct((B, _TK, 1), jnp.int8),
            jax.ShapeDtypeStruct((B, _TK, 1, D), jnp.float32),
        ],
    )(motion_query, plan_query, ego_status_feature, ptm, pem, ete)

    return (out1, out2, out3, out4,
            out5.astype(bool), out6, out7.astype(bool), out8)
